# Initial kernel scaffold; baseline (speedup 1.0000x reference)
import jax
import jax.numpy as jnp
from jax.experimental import pallas as pl


def kernel(h, edge_index, direction, W_fc, W_fcr, W_attn):
    raise NotImplementedError("write your pallas kernel here")


# speedup vs baseline: 11.0667x; 11.0667x over previous
import dataclasses
import functools

import jax
import jax.numpy as jnp
from jax import lax
from jax.experimental import pallas as pl
from jax.experimental.pallas import tpu as pltpu
from jax.experimental.pallas import tpu_sc as plsc

N = 10000
E = 320000
D = 128
L = 16
NC = 2
NS = 16
NW = NC * NS
EPW = E // NW
G = 80
NIT = EPW // G
NPAD = 10240
RPS = NPAD // NS

_mesh = plsc.VectorSubcoreMesh(core_axis_name="c", subcore_axis_name="s")

_cp = pltpu.CompilerParams()
if "needs_layout_passes" in pltpu.CompilerParams.__dataclass_fields__:
    _cp = dataclasses.replace(_cp, needs_layout_passes=False)


def _project_body(h_ref, wfc_ref, wfcr_ref, am_ref, z_ref, zr_ref, s_ref):
    hb = h_ref[...]
    dn = (((1,), (1,)), ((), ()))
    z = lax.dot_general(hb, wfc_ref[...], dn,
                        preferred_element_type=jnp.float32,
                        precision=lax.Precision.HIGHEST)
    zr = lax.dot_general(hb, wfcr_ref[...], dn,
                         preferred_element_type=jnp.float32,
                         precision=lax.Precision.HIGHEST)
    am = am_ref[...]
    dn2 = (((1,), (0,)), ((), ()))
    sz = lax.dot_general(z, am, dn2, preferred_element_type=jnp.float32,
                         precision=lax.Precision.HIGHEST)
    szr = lax.dot_general(zr, am, dn2, preferred_element_type=jnp.float32,
                          precision=lax.Precision.HIGHEST)
    z_ref[...] = z
    zr_ref[...] = zr
    s_ref[...] = jnp.concatenate([sz, szr], axis=1)


def _project(h, W_fc, W_fcr, am):
    R = 1000
    return pl.pallas_call(
        _project_body,
        grid=(N // R,),
        in_specs=[
            pl.BlockSpec((R, D), lambda i: (i, 0)),
            pl.BlockSpec((D, D), lambda i: (0, 0)),
            pl.BlockSpec((D, D), lambda i: (0, 0)),
            pl.BlockSpec((D, 2), lambda i: (0, 0)),
        ],
        out_specs=[
            pl.BlockSpec((R, D), lambda i: (i, 0)),
            pl.BlockSpec((R, D), lambda i: (i, 0)),
            pl.BlockSpec((R, 4), lambda i: (i, 0)),
        ],
        out_shape=[
            jax.ShapeDtypeStruct((N, D), jnp.float32),
            jax.ShapeDtypeStruct((N, D), jnp.float32),
            jax.ShapeDtypeStruct((N, 4), jnp.float32),
        ],
    )(h, W_fc, W_fcr, am)


def _lrelu(x):
    return jnp.where(x >= 0.0, x, 0.01 * x)


@functools.partial(
    pl.kernel,
    mesh=_mesh,
    out_type=[
        jax.ShapeDtypeStruct((E,), jnp.float32),
        jax.ShapeDtypeStruct((NW, L), jnp.float32),
    ],
    scratch_types=[
        pltpu.VMEM((N,), jnp.float32),
        pltpu.VMEM((N,), jnp.float32),
        pltpu.VMEM((N,), jnp.float32),
        pltpu.VMEM((N,), jnp.float32),
        pltpu.VMEM((EPW,), jnp.int32),
        pltpu.VMEM((EPW,), jnp.int32),
        pltpu.VMEM((EPW,), jnp.float32),
        pltpu.VMEM((EPW,), jnp.float32),
        pltpu.VMEM((EPW,), jnp.float32),
        pltpu.VMEM((L,), jnp.float32),
    ],
    compiler_params=_cp,
)
def _edge_scores(s1_hbm, s2_hbm, sr1_hbm, sr2_hbm, src_hbm, dst_hbm,
                 d0_hbm, d1_hbm, ef_hbm, mx_hbm,
                 s1v, s2v, sr1v, sr2v, srcv, dstv, d0v, d1v, efv, mxv):
    cid = lax.axis_index("c")
    sid = lax.axis_index("s")
    wid = cid * NS + sid
    base = wid * EPW
    pltpu.sync_copy(s1_hbm, s1v)
    pltpu.sync_copy(s2_hbm, s2v)
    pltpu.sync_copy(sr1_hbm, sr1v)
    pltpu.sync_copy(sr2_hbm, sr2v)
    pltpu.sync_copy(src_hbm.at[pl.ds(base, EPW)], srcv)
    pltpu.sync_copy(dst_hbm.at[pl.ds(base, EPW)], dstv)
    pltpu.sync_copy(d0_hbm.at[pl.ds(base, EPW)], d0v)
    pltpu.sync_copy(d1_hbm.at[pl.ds(base, EPW)], d1v)
    mxv[...] = jnp.full((L,), -3e38, jnp.float32)

    @pl.loop(0, EPW, step=L)
    def _(g):
        sl = pl.ds(g, L)
        i16 = srcv[sl]
        j16 = dstv[sl]
        a1 = plsc.load_gather(s1v, [i16])
        a2 = plsc.load_gather(s2v, [j16])
        b1 = plsc.load_gather(sr1v, [i16])
        b2 = plsc.load_gather(sr2v, [j16])
        ef16 = d0v[sl] * _lrelu(a1 + a2) + d1v[sl] * _lrelu(b1 + b2)
        efv[sl] = ef16
        mxv[...] = jnp.maximum(mxv[...], ef16)

    pltpu.sync_copy(efv, ef_hbm.at[pl.ds(base, EPW)])
    pltpu.sync_copy(mxv, mx_hbm.at[wid])


@functools.partial(
    pl.kernel,
    mesh=_mesh,
    out_type=[
        jax.ShapeDtypeStruct((NC, NPAD, D), jnp.float32),
        jax.ShapeDtypeStruct((NC, NPAD // 8, D), jnp.float32),
    ],
    scratch_types=[
        pltpu.VMEM((2, G), jnp.int32),
        pltpu.VMEM((2, G), jnp.int32),
        pltpu.VMEM((2, G), jnp.float32),
        pltpu.VMEM((G,), jnp.float32),
        pltpu.VMEM((G, D), jnp.float32),
        pltpu.VMEM((G, D), jnp.float32),
        pltpu.VMEM((G, D), jnp.float32),
        pltpu.VMEM((NW, L), jnp.float32),
        pltpu.VMEM_SHARED((NPAD, D), jnp.float32),
        pltpu.VMEM_SHARED((NPAD // 8, D), jnp.float32),
    ],
    compiler_params=_cp,
)
def _aggregate(z_hbm, zr_hbm, imeta_hbm, fmeta_hbm, ef_hbm,
               mx_hbm, pnum_hbm, pden_hbm,
               imc, idxb, fmc, efc, zrows, zrrows, denb, mxall, snum, sden):
    cid = lax.axis_index("c")
    sid = lax.axis_index("s")
    wid = cid * NS + sid
    base = wid * EPW
    RPS8 = RPS // 8

    z16 = jnp.zeros((L,), jnp.float32)

    @pl.loop(0, G)
    def _(r):
        for j in range(D // L):
            zrows[r, pl.ds(j * L, L)] = z16

    for k in range(RPS // G):
        pltpu.sync_copy(zrows, snum.at[pl.ds(sid * RPS + k * G, G)])
    pltpu.sync_copy(zrows, sden.at[pl.ds(sid * RPS8, RPS8)])
    plsc.subcore_barrier()

    pltpu.sync_copy(mx_hbm, mxall)
    m16 = mxall[0, pl.ds(0, L)]
    for k in range(1, NW):
        m16 = jnp.maximum(m16, mxall[k, pl.ds(0, L)])
    gmax = jnp.max(m16)

    @pl.loop(0, NIT)
    def _(it):
        eb = it * G
        pltpu.sync_copy(imeta_hbm.at[wid, it], imc)
        pltpu.sync_copy(fmeta_hbm.at[wid, it], fmc)
        pltpu.sync_copy(ef_hbm.at[pl.ds(base + eb, G)], efc)
        pltpu.sync_copy(z_hbm.at[imc.at[0]], zrows)
        pltpu.sync_copy(zr_hbm.at[imc.at[0]], zrrows)

        @pl.loop(0, G // L)
        def _(g):
            sl = pl.ds(g * L, L)
            ex16 = jnp.exp(efc[sl] - gmax)
            c0_16 = ex16 * fmc[0, sl]
            c1_16 = ex16 * fmc[1, sl]
            d16 = imc[1, sl]
            idxb[0, sl] = lax.shift_right_logical(d16, 3)
            grp16 = lax.rem(d16, 8)
            for ri in range(L):
                r = g * L + ri
                c0 = c0_16[ri]
                c1 = c1_16[ri]
                exs = ex16[ri]
                grp = grp16[ri]
                for j in range(D // L):
                    sl2 = pl.ds(j * L, L)
                    zrows[r, sl2] = c0 * zrows[r, sl2] + c1 * zrrows[r, sl2]
                    denb[r, sl2] = jnp.where(grp == j, exs, 0.0) + z16

        pltpu.sync_copy(zrows, snum.at[imc.at[1]], add=True)
        pltpu.sync_copy(denb, sden.at[idxb.at[0]], add=True)

    plsc.subcore_barrier()
    pltpu.sync_copy(snum.at[pl.ds(sid * RPS, RPS)],
                    pnum_hbm.at[cid, pl.ds(sid * RPS, RPS)])
    pltpu.sync_copy(sden.at[pl.ds(sid * RPS8, RPS8)],
                    pden_hbm.at[cid, pl.ds(sid * RPS8, RPS8)])


def _finalize_body(pn_ref, pd_ref, o_ref):
    n = pn_ref[0] + pn_ref[1]
    d = pd_ref[0, :, 0:1] + pd_ref[1, :, 0:1]
    o_ref[...] = n / (d + 1e-38)


def _finalize(pnum, pden):
    R = 1024
    return pl.pallas_call(
        _finalize_body,
        grid=(NPAD // R,),
        in_specs=[
            pl.BlockSpec((NC, R, D), lambda i: (0, i, 0)),
            pl.BlockSpec((NC, R, L), lambda i: (0, i, 0)),
        ],
        out_specs=pl.BlockSpec((R, D), lambda i: (i, 0)),
        out_shape=jax.ShapeDtypeStruct((NPAD, D), jnp.float32),
    )(pnum, pden)


def kernel(h, edge_index, direction, W_fc, W_fcr, W_attn):
    src = edge_index[0].astype(jnp.int32)
    dst = edge_index[1].astype(jnp.int32)
    d0 = direction[:, 0, 0]
    d1 = direction[:, 1, 0]
    am = W_attn.reshape(2, D).T

    z, zr, s = _project(h, W_fc, W_fcr, am)
    s1 = s[:, 0]
    s2 = s[:, 1]
    sr1 = s[:, 2]
    sr2 = s[:, 3]

    ef, mx = _edge_scores(s1, s2, sr1, sr2, src, dst, d0, d1)

    imeta = jnp.stack([src, dst], axis=0).reshape(2, NW, NIT, G).transpose(1, 2, 0, 3)
    fmeta = jnp.stack([d0, d1], axis=0).reshape(2, NW, NIT, G).transpose(1, 2, 0, 3)
    pnum, pden = _aggregate(z, zr, imeta, fmeta, ef, mx)

    out = _finalize(pnum, pden.reshape(NC, NPAD, L))
    return out[:N]
